# baseline (speedup 1.0000x reference)
import jax
import jax.numpy as jnp
from jax.experimental import pallas as pl
from jax.experimental.pallas import tpu as pltpu

_ROWS = 8


def _gather_body(idx_ref, *refs):
    out_ref = refs[_ROWS]
    for j in range(_ROWS):
        out_ref[j] = refs[j][0]


def kernel(input, weight):
    bptt, bsize = input.shape
    vocab, emsize = weight.shape
    n = bptt * bsize
    idx = input.reshape(n).astype(jnp.int32)
    w3 = weight.reshape(vocab, 1, emsize)
    in_specs = [
        pl.BlockSpec((1, 1, emsize),
                     (lambda i, idx_ref, j=j: (idx_ref[i * _ROWS + j], 0, 0)))
        for j in range(_ROWS)
    ]
    out = pl.pallas_call(
        _gather_body,
        grid_spec=pltpu.PrefetchScalarGridSpec(
            num_scalar_prefetch=1,
            grid=(n // _ROWS,),
            in_specs=in_specs,
            out_specs=pl.BlockSpec((_ROWS, 1, emsize),
                                   lambda i, idx_ref: (i, 0, 0)),
        ),
        out_shape=jax.ShapeDtypeStruct((n, 1, emsize), weight.dtype),
        compiler_params=pltpu.CompilerParams(
            dimension_semantics=("arbitrary",)),
        name="embedding_gather",
    )(idx, *([w3] * _ROWS))
    return out.reshape(bptt, bsize, emsize)


# speedup vs baseline: 4.0748x; 4.0748x over previous
import jax
import jax.numpy as jnp
from jax.experimental import pallas as pl
from jax.experimental.pallas import tpu as pltpu

_BLK = 512
_UNROLL = 8


def _gather_body(idx_ref, w_ref, out_ref, sem):
    base = pl.program_id(0) * _BLK

    def issue(c, carry):
        b = base + c * _UNROLL
        for u in range(_UNROLL):
            r = idx_ref[b + u]
            pltpu.make_async_copy(
                w_ref.at[pl.ds(r, 1)],
                out_ref.at[pl.ds(c * _UNROLL + u, 1)],
                sem,
            ).start()
        return carry

    jax.lax.fori_loop(0, _BLK // _UNROLL, issue, 0)
    pltpu.make_async_copy(w_ref.at[pl.ds(0, _BLK)], out_ref, sem).wait()


def kernel(input, weight):
    bptt, bsize = input.shape
    vocab, emsize = weight.shape
    n = bptt * bsize
    idx = input.reshape(n).astype(jnp.int32)
    w3 = weight.reshape(vocab, 1, emsize)
    out = pl.pallas_call(
        _gather_body,
        grid_spec=pltpu.PrefetchScalarGridSpec(
            num_scalar_prefetch=1,
            grid=(n // _BLK,),
            in_specs=[pl.BlockSpec(memory_space=pl.ANY)],
            out_specs=pl.BlockSpec((_BLK, 1, emsize),
                                   lambda i, idx_ref: (i, 0, 0)),
            scratch_shapes=[pltpu.SemaphoreType.DMA],
        ),
        out_shape=jax.ShapeDtypeStruct((n, 1, emsize), weight.dtype),
        compiler_params=pltpu.CompilerParams(
            dimension_semantics=("arbitrary",)),
        name="embedding_gather",
    )(idx, w3)
    return out.reshape(bptt, bsize, emsize)


# speedup vs baseline: 4.1791x; 1.0256x over previous
import jax
import jax.numpy as jnp
from jax.experimental import pallas as pl
from jax.experimental.pallas import tpu as pltpu

_BLK = 512
_UNROLL = 8


def _gather_body(idx_ref, w_ref, out_ref, sem):
    base = pl.program_id(0) * _BLK

    def issue(c, carry):
        b = base + c * _UNROLL
        for u in range(_UNROLL):
            r = idx_ref[b + u]
            pltpu.make_async_copy(
                w_ref.at[pl.ds(r, 1)],
                out_ref.at[pl.ds(c * _UNROLL + u, 1)],
                sem,
            ).start(priority=u % 2)
        return carry

    jax.lax.fori_loop(0, _BLK // _UNROLL, issue, 0)
    pltpu.make_async_copy(w_ref.at[pl.ds(0, _BLK)], out_ref, sem).wait()


def kernel(input, weight):
    bptt, bsize = input.shape
    vocab, emsize = weight.shape
    n = bptt * bsize
    idx = input.reshape(n).astype(jnp.int32)
    w3 = weight.reshape(vocab, 1, emsize)
    out = pl.pallas_call(
        _gather_body,
        grid_spec=pltpu.PrefetchScalarGridSpec(
            num_scalar_prefetch=1,
            grid=(n // _BLK,),
            in_specs=[pl.BlockSpec(memory_space=pl.ANY)],
            out_specs=pl.BlockSpec((_BLK, 1, emsize),
                                   lambda i, idx_ref: (i, 0, 0)),
            scratch_shapes=[pltpu.SemaphoreType.DMA],
        ),
        out_shape=jax.ShapeDtypeStruct((n, 1, emsize), weight.dtype),
        compiler_params=pltpu.CompilerParams(
            dimension_semantics=("arbitrary",)),
        name="embedding_gather",
    )(idx, w3)
    return out.reshape(bptt, bsize, emsize)
